# baseline (speedup 1.0000x reference)
import functools

import jax
import jax.numpy as jnp
from jax import lax
from jax.experimental import pallas as pl
from jax.experimental.pallas import tpu as pltpu
from jax.experimental.pallas import tpu_sc as plsc

_TOPK = 64
_ROWS = 4096
_COLS = 16384
_NC = 2
_NS = 16
_NW = _NC * _NS
_RPW = _ROWS // _NW
_NSLAB = _RPW // 8
_CHC = _COLS // 4
_MARGIN = 1 << 20
_CCAP = 2048
_CROW = _CCAP + 64
_MROWS = 128
_IMIN = -(2 ** 31)


def _key16(v):
    u = plsc.bitcast(v, jnp.int32)
    return u ^ (jnp.right_shift(u, 31) & jnp.int32(0x7FFFFFFF))


def _count_keys(load, n4, t):
    def body(i, acc):
        base = i * 64
        for u in range(4):
            acc = acc + plsc.all_reduce_population_count(load(base + u * 16) >= t)
        return acc
    return lax.fori_loop(0, n4, body, jnp.zeros((16,), jnp.int32))[0]


def _bisect(load, n4):
    cpos = _count_keys(load, n4, jnp.int32(0))
    t = jnp.where(cpos >= _TOPK, jnp.int32(0), jnp.int32(_IMIN))

    def bit_body(j, tt):
        cand = tt | (jnp.int32(1) << (30 - j))
        cc = _count_keys(load, n4, cand)
        return jnp.where(cc >= _TOPK, cand, tt)
    return lax.fori_loop(0, 31, bit_body, t)


def _sc_body(x_hbm, out_hbm, bufa_v, bufb_v, frow_v, cand_v, thr_v, sem):
    wid = lax.axis_index("s") * _NC + lax.axis_index("c")
    lane = lax.iota(jnp.int32, 16)
    r0 = wid * _RPW
    pad = jnp.full((16,), _IMIN, jnp.int32)

    def full_row_threshold(r):
        pltpu.sync_copy(x_hbm.at[pl.ds(r, 1)], frow_v)

        def cp(i, _):
            bufb_v[i // 256, pl.ds((i % 256) * 16, 16)] = plsc.bitcast(
                _key16(frow_v[0, pl.ds(i * 16, 16)]), jnp.float32)
            return 0
        lax.fori_loop(0, _COLS // 16, cp, 0)
        for u in range(4):
            bufb_v[4, pl.ds(u * 16, 16)] = plsc.bitcast(pad, jnp.float32)

        def ld(i):
            return plsc.bitcast(bufb_v[i // _CHC, pl.ds(i % _CHC, 16)],
                                jnp.int32)
        return _bisect(ld, _COLS // 64 + 1)

    def filter_chunk_all(buf, t0, offv):
        def jb(j, offv):
            off0 = jnp.sum(jnp.where(lane == j, offv, jnp.int32(0)))

            def fb(i, off):
                for u2 in range(2):
                    v = 2 * i + u2
                    kv = _key16(buf[j, pl.ds(v * 16, 16)])
                    m = kv >= t0
                    pc = plsc.all_reduce_population_count(m)[0]
                    mv = jnp.logical_and(m, off <= _CCAP - 16)
                    off_st = jnp.minimum(off, _CCAP)
                    plsc.store_compressed(
                        cand_v.at[pl.ds(j * _CROW + off_st, 16)],
                        kv, mask=mv)
                    off = off + pc
                return off
            off = lax.fori_loop(0, _CHC // 32, fb, off0)
            return jnp.where(lane == j, off, offv)
        return lax.fori_loop(0, 8, jb, offv)

    def slab_body(s, carry):
        t_prev, tvec = carry
        rs = r0 + 8 * s
        t0 = t_prev - jnp.int32(_MARGIN)
        t0 = jnp.where(t0 > t_prev, jnp.int32(_IMIN), t0)

        offv = jnp.zeros((16,), jnp.int32)
        for k in range(4):
            buf = bufa_v if k % 2 == 0 else bufb_v
            other = bufb_v if k % 2 == 0 else bufa_v
            pltpu.make_async_copy(
                x_hbm.at[pl.ds(rs, 8), pl.ds(k * _CHC, _CHC)], buf, sem).wait()
            if k < 3:
                pltpu.async_copy(
                    x_hbm.at[pl.ds(rs, 8), pl.ds((k + 1) * _CHC, _CHC)],
                    other, sem)
            else:
                @pl.when(s < _NSLAB - 1)
                def _():
                    pltpu.async_copy(
                        x_hbm.at[pl.ds(rs + 8, 8), pl.ds(0, _CHC)],
                        other, sem)
            offv = filter_chunk_all(buf, t0, offv)

        def fin(j, carry):
            tp, tvec = carry
            c = jnp.sum(jnp.where(lane == j, offv, jnp.int32(0)))
            ok = jnp.logical_and(c >= _TOPK, c <= _CCAP - 16)

            def normal():
                for u in range(4):
                    cand_v[pl.ds(j * _CROW + c + u * 16, 16)] = pad
                return _bisect(lambda i: cand_v[pl.ds(j * _CROW + i, 16)],
                               (c + 63) // 64)

            def fallback():
                return full_row_threshold(rs + j)

            t = lax.cond(ok, normal, fallback)
            tvec = jnp.where(lane == (8 * s + j) % 16, t, tvec)
            return t, tvec

        tp, tvec = lax.fori_loop(0, 8, fin, (t_prev, tvec))

        @pl.when(s % 2 == 1)
        def _():
            thr_v[pl.ds((s // 2) * 16, 16)] = tvec

        return tp, tvec

    t_boot = full_row_threshold(r0)
    pltpu.async_copy(x_hbm.at[pl.ds(r0, 8), pl.ds(0, _CHC)], bufa_v, sem)
    lax.fori_loop(0, _NSLAB, slab_body,
                  (t_boot, jnp.zeros((16,), jnp.int32)))
    pltpu.sync_copy(thr_v, out_hbm.at[pl.ds(r0, _RPW)])


_sc_thresholds = functools.partial(
    pl.kernel,
    mesh=plsc.VectorSubcoreMesh(core_axis_name="c", subcore_axis_name="s"),
    out_type=jax.ShapeDtypeStruct((_ROWS,), jnp.int32),
    scratch_types=[
        pltpu.VMEM((8, _CHC), jnp.float32),
        pltpu.VMEM((8, _CHC), jnp.float32),
        pltpu.VMEM((1, _COLS), jnp.float32),
        pltpu.VMEM((8 * _CROW,), jnp.int32),
        pltpu.VMEM((_RPW,), jnp.int32),
        pltpu.SemaphoreType.DMA,
    ],
    compiler_params=pltpu.CompilerParams(needs_layout_passes=False),
)(_sc_body)


def _mask_block(x_ref, t_ref, o_ref):
    x = x_ref[...]
    tau = t_ref[...]
    o_ref[...] = jnp.where(x >= tau, x, jnp.float32(0.0))


def kernel(inputs):
    x = inputs
    tk = _sc_thresholds(x)
    bits = jnp.where(tk >= 0, tk, tk ^ jnp.int32(0x7FFFFFFF))
    tau = lax.bitcast_convert_type(bits, jnp.float32).reshape(_ROWS, 1)
    return pl.pallas_call(
        _mask_block,
        grid=(_ROWS // _MROWS,),
        in_specs=[
            pl.BlockSpec((_MROWS, _COLS), lambda i: (i, 0)),
            pl.BlockSpec((_MROWS, 1), lambda i: (i, 0)),
        ],
        out_specs=pl.BlockSpec((_MROWS, _COLS), lambda i: (i, 0)),
        out_shape=jax.ShapeDtypeStruct((_ROWS, _COLS), jnp.float32),
        compiler_params=pltpu.CompilerParams(
            dimension_semantics=("arbitrary",)),
    )(x, tau)


# speedup vs baseline: 1.2043x; 1.2043x over previous
import functools

import jax
import jax.numpy as jnp
from jax import lax
from jax.experimental import pallas as pl
from jax.experimental.pallas import tpu as pltpu
from jax.experimental.pallas import tpu_sc as plsc

_TOPK = 64
_ROWS = 4096
_COLS = 16384
_NC = 2
_NS = 16
_NW = _NC * _NS
_RPW = _ROWS // _NW
_NSLAB = _RPW // 8
_CHC = _COLS // 4
_MARGIN = 1 << 20
_CCAP = 2048
_CROW = _CCAP + 64
_MROWS = 128
_IMIN = -(2 ** 31)


def _key16(v):
    u = plsc.bitcast(v, jnp.int32)
    return u ^ (jnp.right_shift(u, 31) & jnp.int32(0x7FFFFFFF))


def _count_keys(load, n4, t):
    def body(i, acc):
        base = i * 64
        for u in range(4):
            acc = acc + plsc.all_reduce_population_count(load(base + u * 16) >= t)
        return acc
    return lax.fori_loop(0, n4, body, jnp.zeros((16,), jnp.int32))[0]


def _bisect(load, n4):
    cpos = _count_keys(load, n4, jnp.int32(0))
    t = jnp.where(cpos >= _TOPK, jnp.int32(0), jnp.int32(_IMIN))

    def bit_body(j, tt):
        cand = tt | (jnp.int32(1) << (30 - j))
        cc = _count_keys(load, n4, cand)
        return jnp.where(cc >= _TOPK, cand, tt)
    return lax.fori_loop(0, 31, bit_body, t)


def _sc_body(x_hbm, out_hbm, bufa_v, bufb_v, frow_v, cand_v, thr_v, sem):
    wid = lax.axis_index("s") * _NC + lax.axis_index("c")
    lane = lax.iota(jnp.int32, 16)
    r0 = wid * _RPW
    pad = jnp.full((16,), _IMIN, jnp.int32)

    def full_row_threshold(r):
        pltpu.sync_copy(x_hbm.at[pl.ds(r, 1)], frow_v)

        def cp(i, _):
            bufb_v[i // 256, pl.ds((i % 256) * 16, 16)] = plsc.bitcast(
                _key16(frow_v[0, pl.ds(i * 16, 16)]), jnp.float32)
            return 0
        lax.fori_loop(0, _COLS // 16, cp, 0)
        for u in range(4):
            bufb_v[4, pl.ds(u * 16, 16)] = plsc.bitcast(pad, jnp.float32)

        def ld(i):
            return plsc.bitcast(bufb_v[i // _CHC, pl.ds(i % _CHC, 16)],
                                jnp.int32)
        return _bisect(ld, _COLS // 64 + 1)

    def filter_chunk_all(buf, t0, offv):
        def jb(j, offv):
            off0 = jnp.sum(jnp.where(lane == j, offv, jnp.int32(0)))

            def fb(i, off):
                for u2 in range(4):
                    v = 4 * i + u2
                    kv = _key16(buf[j, pl.ds(v * 16, 16)])
                    m = kv >= t0
                    pc = plsc.all_reduce_population_count(m)[0]
                    off_st = off & jnp.int32(_CCAP - 1)
                    plsc.store_compressed(
                        cand_v.at[pl.ds(j * _CROW + off_st, 16)],
                        kv, mask=m)
                    off = off + pc
                return off
            off = lax.fori_loop(0, _CHC // 64, fb, off0)
            return jnp.where(lane == j, off, offv)
        return lax.fori_loop(0, 8, jb, offv)

    def slab_body(s, carry):
        t_prev, tvec = carry
        rs = r0 + 8 * s
        t0 = t_prev - jnp.int32(_MARGIN)
        t0 = jnp.where(t0 > t_prev, jnp.int32(_IMIN), t0)

        offv = jnp.zeros((16,), jnp.int32)
        for k in range(4):
            buf = bufa_v if k % 2 == 0 else bufb_v
            other = bufb_v if k % 2 == 0 else bufa_v
            pltpu.make_async_copy(
                x_hbm.at[pl.ds(rs, 8), pl.ds(k * _CHC, _CHC)], buf, sem).wait()
            if k < 3:
                pltpu.async_copy(
                    x_hbm.at[pl.ds(rs, 8), pl.ds((k + 1) * _CHC, _CHC)],
                    other, sem)
            else:
                @pl.when(s < _NSLAB - 1)
                def _():
                    pltpu.async_copy(
                        x_hbm.at[pl.ds(rs + 8, 8), pl.ds(0, _CHC)],
                        other, sem)
            offv = filter_chunk_all(buf, t0, offv)

        def fin(j, carry):
            tp, tvec = carry
            c = jnp.sum(jnp.where(lane == j, offv, jnp.int32(0)))
            ok = jnp.logical_and(c >= _TOPK, c <= _CCAP - 16)

            def normal():
                for u in range(4):
                    cand_v[pl.ds(j * _CROW + c + u * 16, 16)] = pad
                return _bisect(lambda i: cand_v[pl.ds(j * _CROW + i, 16)],
                               (c + 63) // 64)

            def fallback():
                return full_row_threshold(rs + j)

            t = lax.cond(ok, normal, fallback)
            tvec = jnp.where(lane == (8 * s + j) % 16, t, tvec)
            return t, tvec

        tp, tvec = lax.fori_loop(0, 8, fin, (t_prev, tvec))

        @pl.when(s % 2 == 1)
        def _():
            thr_v[pl.ds((s // 2) * 16, 16)] = tvec

        return tp, tvec

    t_boot = full_row_threshold(r0)
    pltpu.async_copy(x_hbm.at[pl.ds(r0, 8), pl.ds(0, _CHC)], bufa_v, sem)
    lax.fori_loop(0, _NSLAB, slab_body,
                  (t_boot, jnp.zeros((16,), jnp.int32)))
    pltpu.sync_copy(thr_v, out_hbm.at[pl.ds(r0, _RPW)])


_sc_thresholds = functools.partial(
    pl.kernel,
    mesh=plsc.VectorSubcoreMesh(core_axis_name="c", subcore_axis_name="s"),
    out_type=jax.ShapeDtypeStruct((_ROWS,), jnp.int32),
    scratch_types=[
        pltpu.VMEM((8, _CHC), jnp.float32),
        pltpu.VMEM((8, _CHC), jnp.float32),
        pltpu.VMEM((1, _COLS), jnp.float32),
        pltpu.VMEM((8 * _CROW,), jnp.int32),
        pltpu.VMEM((_RPW,), jnp.int32),
        pltpu.SemaphoreType.DMA,
    ],
    compiler_params=pltpu.CompilerParams(needs_layout_passes=False),
)(_sc_body)


def _mask_block(x_ref, t_ref, o_ref):
    x = x_ref[...]
    tau = t_ref[...]
    o_ref[...] = jnp.where(x >= tau, x, jnp.float32(0.0))


def kernel(inputs):
    x = inputs
    tk = _sc_thresholds(x)
    bits = jnp.where(tk >= 0, tk, tk ^ jnp.int32(0x7FFFFFFF))
    tau = lax.bitcast_convert_type(bits, jnp.float32).reshape(_ROWS, 1)
    return pl.pallas_call(
        _mask_block,
        grid=(_ROWS // _MROWS,),
        in_specs=[
            pl.BlockSpec((_MROWS, _COLS), lambda i: (i, 0)),
            pl.BlockSpec((_MROWS, 1), lambda i: (i, 0)),
        ],
        out_specs=pl.BlockSpec((_MROWS, _COLS), lambda i: (i, 0)),
        out_shape=jax.ShapeDtypeStruct((_ROWS, _COLS), jnp.float32),
        compiler_params=pltpu.CompilerParams(
            dimension_semantics=("arbitrary",)),
    )(x, tau)
